# baseline (speedup 1.0000x reference)
import jax
import jax.numpy as jnp
from jax.experimental import pallas as pl
from jax.experimental.pallas import tpu as pltpu

NPOS = 8192
NPRIM = 1024
NORB = 256
BN = 512

_LOG2E = 1.4426950408889634


def _basis_block(pos_ref, cn_ref, centerT_ref, at_ref, lmnT_ref, oi_ref,
                 out_ref, s_ref):
    @pl.when(pl.program_id(0) == 0)
    def _build_s():
        col = jax.lax.broadcasted_iota(jnp.int32, (NORB, NPRIM), 0)
        s_ref[...] = jnp.where(col == oi_ref[...], cn_ref[...], 0.0)

    p = pos_ref[...]
    x = p[:, 0:1]
    y = p[:, 1:2]
    z = p[:, 2:3]

    cx = centerT_ref[0:1, :]
    cy = centerT_ref[1:2, :]
    cz = centerT_ref[2:3, :]

    dx = x - cx
    dy = y - cy
    dz = z - cz
    d2x = dx * dx
    d2y = dy * dy
    d2z = dz * dz
    r2 = (d2x + d2y) + d2z

    lx = lmnT_ref[0:1, :]
    ly = lmnT_ref[1:2, :]
    lz = lmnT_ref[2:3, :]
    ax = jnp.where(lx == 0, 1.0, jnp.where(lx == 1, dx, d2x))
    ay = jnp.where(ly == 0, 1.0, jnp.where(ly == 1, dy, d2y))
    az = jnp.where(lz == 0, 1.0, jnp.where(lz == 1, dz, d2z))

    ex = jnp.exp2(at_ref[...] * r2)
    prim = (ax * ay) * (az * ex)

    out_ref[...] = jax.lax.dot_general(
        prim, s_ref[...], (((1,), (1,)), ((), ())),
        preferred_element_type=jnp.float32)


@jax.jit
def kernel(pos, coefficients, center, alpha, norm, lmn, orbital_index):
    cn = (coefficients * norm).reshape(1, NPRIM)
    centerT = center.T
    lmnT = lmn.T
    at = (-_LOG2E * alpha).reshape(1, NPRIM)
    oi = orbital_index.reshape(1, NPRIM)

    grid = (NPOS // BN,)
    return pl.pallas_call(
        _basis_block,
        grid=grid,
        in_specs=[
            pl.BlockSpec((BN, 3), lambda i: (i, 0)),
            pl.BlockSpec((1, NPRIM), lambda i: (0, 0)),
            pl.BlockSpec((3, NPRIM), lambda i: (0, 0)),
            pl.BlockSpec((1, NPRIM), lambda i: (0, 0)),
            pl.BlockSpec((3, NPRIM), lambda i: (0, 0)),
            pl.BlockSpec((1, NPRIM), lambda i: (0, 0)),
        ],
        out_specs=pl.BlockSpec((BN, NORB), lambda i: (i, 0)),
        out_shape=jax.ShapeDtypeStruct((NPOS, NORB), jnp.float32),
        scratch_shapes=[pltpu.VMEM((NORB, NPRIM), jnp.float32)],
        compiler_params=pltpu.CompilerParams(
            dimension_semantics=("arbitrary",)),
    )(pos, cn, centerT, at, lmnT, oi)


# speedup vs baseline: 1.0352x; 1.0352x over previous
import jax
import jax.numpy as jnp
from jax.experimental import pallas as pl
from jax.experimental.pallas import tpu as pltpu

NPOS = 8192
NPRIM = 1024
NORB = 256
BN = 1024

_LOG2E = 1.4426950408889634


def _basis_block(pos_ref, cn_ref, centerT_ref, at_ref, lmnT_ref, oi_ref,
                 out_ref, s_ref):
    @pl.when(pl.program_id(0) == 0)
    def _build_s():
        col = jax.lax.broadcasted_iota(jnp.int32, (NORB, NPRIM), 0)
        s_ref[...] = jnp.where(col == oi_ref[...], cn_ref[...], 0.0)

    p = pos_ref[...]
    x = p[:, 0:1]
    y = p[:, 1:2]
    z = p[:, 2:3]

    cx = centerT_ref[0:1, :]
    cy = centerT_ref[1:2, :]
    cz = centerT_ref[2:3, :]

    dx = x - cx
    dy = y - cy
    dz = z - cz
    d2x = dx * dx
    d2y = dy * dy
    d2z = dz * dz
    r2 = (d2x + d2y) + d2z

    lx = lmnT_ref[0:1, :]
    ly = lmnT_ref[1:2, :]
    lz = lmnT_ref[2:3, :]
    ax = jnp.where(lx == 0, 1.0, jnp.where(lx == 1, dx, d2x))
    ay = jnp.where(ly == 0, 1.0, jnp.where(ly == 1, dy, d2y))
    az = jnp.where(lz == 0, 1.0, jnp.where(lz == 1, dz, d2z))

    ex = jnp.exp2(at_ref[...] * r2)
    prim = (ax * ay) * (az * ex)

    out_ref[...] = jax.lax.dot_general(
        prim, s_ref[...], (((1,), (1,)), ((), ())),
        preferred_element_type=jnp.float32)


@jax.jit
def kernel(pos, coefficients, center, alpha, norm, lmn, orbital_index):
    cn = (coefficients * norm).reshape(1, NPRIM)
    centerT = center.T
    lmnT = lmn.T
    at = (-_LOG2E * alpha).reshape(1, NPRIM)
    oi = orbital_index.reshape(1, NPRIM)

    grid = (NPOS // BN,)
    return pl.pallas_call(
        _basis_block,
        grid=grid,
        in_specs=[
            pl.BlockSpec((BN, 3), lambda i: (i, 0)),
            pl.BlockSpec((1, NPRIM), lambda i: (0, 0)),
            pl.BlockSpec((3, NPRIM), lambda i: (0, 0)),
            pl.BlockSpec((1, NPRIM), lambda i: (0, 0)),
            pl.BlockSpec((3, NPRIM), lambda i: (0, 0)),
            pl.BlockSpec((1, NPRIM), lambda i: (0, 0)),
        ],
        out_specs=pl.BlockSpec((BN, NORB), lambda i: (i, 0)),
        out_shape=jax.ShapeDtypeStruct((NPOS, NORB), jnp.float32),
        scratch_shapes=[pltpu.VMEM((NORB, NPRIM), jnp.float32)],
        compiler_params=pltpu.CompilerParams(
            dimension_semantics=("arbitrary",)),
    )(pos, cn, centerT, at, lmnT, oi)
